# baseline (speedup 1.0000x reference)
import functools

import jax
import jax.numpy as jnp
from jax import lax
from jax.experimental import pallas as pl
from jax.experimental.pallas import tpu as pltpu
from jax.experimental.pallas import tpu_sc as plsc

VOCAB = 1000000
DIM = 16
NBAGS = 16384
NTOT = 819200

NW = 32
BAGS_W = NBAGS // NW
CH = 512
NGSUB = CH // 128

OFFS_PAD = NBAGS + BAGS_W + 16
IDX_PAD = NTOT + CH

GROUPS_PER_ROW = 128 // DIM
GVOCAB = VOCAB // GROUPS_PER_ROW
GBLK = 1000


def _deq_body(t_ref, o_ref):
    x = t_ref[...]
    lane = jax.lax.broadcasted_iota(jnp.int32, x.shape, 1)
    mn = x
    mx = x
    for k in (8, 4, 2, 1):
        sel = (lane & k) == 0
        mn = jnp.minimum(
            mn, jnp.where(sel, pltpu.roll(mn, 128 - k, 1),
                          pltpu.roll(mn, k, 1)))
        mx = jnp.maximum(
            mx, jnp.where(sel, pltpu.roll(mx, 128 - k, 1),
                          pltpu.roll(mx, k, 1)))
    scale = (mx - mn) * (1.0 / 255.0)
    scale = jnp.where(scale <= 0.0, jnp.float32(1e-8), scale)
    q = jnp.clip(jnp.round((x - mn) / scale), 0.0, 255.0)
    o_ref[...] = q * scale + mn


def _dequant(table):
    tg = table.reshape(GVOCAB, 128)
    deq = pl.pallas_call(
        _deq_body,
        grid=(GVOCAB // GBLK,),
        in_specs=[pl.BlockSpec((GBLK, 128), lambda i: (i, 0))],
        out_specs=pl.BlockSpec((GBLK, 128), lambda i: (i, 0)),
        out_shape=jax.ShapeDtypeStruct((GVOCAB, 128), jnp.float32),
    )(tg)
    return deq.reshape(VOCAB, DIM)


@functools.partial(
    pl.kernel,
    out_type=jax.ShapeDtypeStruct((NBAGS, DIM), jnp.float32),
    mesh=plsc.VectorSubcoreMesh(core_axis_name="c", subcore_axis_name="s"),
    compiler_params=pltpu.CompilerParams(use_tc_tiling_on_sc=False),
    scratch_types=[
        pltpu.VMEM((BAGS_W + 16,), jnp.int32),
        pltpu.VMEM((CH,), jnp.int32),
        pltpu.VMEM((CH,), jnp.int32),
        pltpu.VMEM((CH, DIM), jnp.float32),
        pltpu.VMEM((CH, DIM), jnp.float32),
        pltpu.VMEM((BAGS_W, DIM), jnp.float32),
        pltpu.VMEM((DIM,), jnp.float32),
        pltpu.SMEM((8,), jnp.int32),
        pltpu.SemaphoreType.DMA,
        pltpu.SemaphoreType.DMA,
    ],
)
def _bag_kernel(offs_hbm, idx_hbm, deq_hbm, out_hbm,
                offs_v, idx_a, idx_b, rows_a, rows_b, out_v, acc_v,
                bcur_s, sem_a, sem_b):
    wid = lax.axis_index("s") * 2 + lax.axis_index("c")
    bag_lo = wid * BAGS_W
    pltpu.sync_copy(offs_hbm.at[pl.ds(bag_lo, BAGS_W + 16)], offs_v)
    start = offs_v[pl.ds(0, 16)][0]
    end = offs_v[pl.ds(BAGS_W, 16)][0]
    abase = (start // 8) * 8
    nch = (end - abase + CH - 1) // CH

    zero = jnp.zeros((DIM,), jnp.float32)
    acc_v[...] = zero
    bcur_s[0] = jnp.int32(0)

    def _zero_body(i, _):
        out_v[i, :] = zero
        return 0
    lax.fori_loop(0, BAGS_W, _zero_body, 0)

    def _fetch(ci, idx_v, rows_v, sem):
        cb = abase + ci * CH
        pltpu.sync_copy(idx_hbm.at[pl.ds(cb, CH)], idx_v)
        for j in range(NGSUB):
            pltpu.make_async_copy(
                deq_hbm.at[idx_v.at[pl.ds(j * 128, 128)]],
                rows_v.at[pl.ds(j * 128, 128)], sem).start()

    def _drain(idx_v, rows_v, sem):
        for j in range(NGSUB):
            pltpu.make_async_copy(
                deq_hbm.at[idx_v.at[pl.ds(j * 128, 128)]],
                rows_v.at[pl.ds(j * 128, 128)], sem).wait()

    def _walk(ci, rows_v):
        cb = abase + ci * CH
        ce = cb + CH

        def _bstep(i, cnt):
            nc = cnt + (BAGS_W >> i)
            pi = jnp.minimum(nc, BAGS_W)
            probe = offs_v[pl.ds(pi, 16)][0]
            ok = jnp.logical_and(nc <= BAGS_W, probe <= ce)
            return jnp.where(ok, nc, cnt)
        cnt = lax.fori_loop(0, 10, _bstep, jnp.int32(0))

        bcur0 = bcur_s[0]
        nb_closed = jnp.maximum(cnt - bcur0, 0)
        extra = jnp.where(bcur0 + nb_closed < BAGS_W, 1, 0)
        trips = nb_closed + extra

        def _bag_body(t, b):
            bv = offs_v[pl.ds(b, 16)]
            bs = bv[0]
            be = bv[1]
            ps = jnp.maximum(bs, cb)
            pe = jnp.minimum(be, ce)
            n = pe - ps
            l0 = ps - cb

            def _add8(j, accs):
                p = l0 + j * 8
                return tuple(accs[i] + rows_v[p + i, :] for i in range(8))
            accs = lax.fori_loop(0, n >> 3, _add8,
                                 (zero,) * 8)
            t8 = ((accs[0] + accs[1]) + (accs[2] + accs[3])) + \
                 ((accs[4] + accs[5]) + (accs[6] + accs[7]))

            def _add1(p, a):
                return a + rows_v[p, :]
            tail = lax.fori_loop(l0 + (n & ~7), l0 + n, _add1, t8)

            total = acc_v[...] + tail
            closed = be <= ce

            @pl.when(closed)
            def _():
                out_v[b, :] = total
                acc_v[...] = zero

            @pl.when(jnp.logical_not(closed))
            def _():
                acc_v[...] = total

            return jnp.where(closed, b + 1, b)

        bcur_s[0] = lax.fori_loop(0, trips, _bag_body, bcur0)

    @pl.when(nch > 0)
    def _():
        _fetch(0, idx_a, rows_a, sem_a)

    def _pair_body(t, _):
        ci0 = 2 * t
        ci1 = ci0 + 1

        @pl.when(ci0 < nch)
        def _():
            @pl.when(ci1 < nch)
            def _():
                _fetch(ci1, idx_b, rows_b, sem_b)
            _drain(idx_a, rows_a, sem_a)
            _walk(ci0, rows_a)

        @pl.when(ci1 < nch)
        def _():
            @pl.when(ci1 + 1 < nch)
            def _():
                _fetch(ci1 + 1, idx_a, rows_a, sem_a)
            _drain(idx_b, rows_b, sem_b)
            _walk(ci1, rows_b)

        return 0

    lax.fori_loop(0, (nch + 1) // 2, _pair_body, 0)
    pltpu.sync_copy(out_v, out_hbm.at[pl.ds(bag_lo, BAGS_W)])


def kernel(table, indices, offsets):
    deq = _dequant(table)
    offs_p = jnp.concatenate(
        [offsets.astype(jnp.int32),
         jnp.full((OFFS_PAD - (NBAGS + 1),), NTOT, jnp.int32)])
    idx_p = jnp.concatenate(
        [indices.astype(jnp.int32), jnp.zeros((IDX_PAD - NTOT,), jnp.int32)])
    return _bag_kernel(offs_p, idx_p, deq)


# speedup vs baseline: 1.2416x; 1.2416x over previous
import functools

import jax
import jax.numpy as jnp
from jax import lax
from jax.experimental import pallas as pl
from jax.experimental.pallas import tpu as pltpu
from jax.experimental.pallas import tpu_sc as plsc

VOCAB = 1000000
DIM = 16
NBAGS = 16384
NTOT = 819200

NW = 32
BAGS_W = NBAGS // NW
CH = 512
NGSUB = CH // 128

OFFS_PAD = NBAGS + BAGS_W + 16
IDX_PAD = NTOT + CH

GROUPS_PER_ROW = 128 // DIM
GVOCAB = VOCAB // GROUPS_PER_ROW
GBLK = 1000


def _deq_body(t_ref, o_ref):
    x = t_ref[...]
    lane = jax.lax.broadcasted_iota(jnp.int32, x.shape, 1)
    mn = x
    mx = x
    for k in (8, 4, 2, 1):
        sel = (lane & k) == 0
        mn = jnp.minimum(
            mn, jnp.where(sel, pltpu.roll(mn, 128 - k, 1),
                          pltpu.roll(mn, k, 1)))
        mx = jnp.maximum(
            mx, jnp.where(sel, pltpu.roll(mx, 128 - k, 1),
                          pltpu.roll(mx, k, 1)))
    scale = (mx - mn) * (1.0 / 255.0)
    scale = jnp.where(scale <= 0.0, jnp.float32(1e-8), scale)
    q = jnp.clip(jnp.round((x - mn) / scale), 0.0, 255.0)
    o_ref[...] = q * scale + mn


def _dequant(table):
    tg = table.reshape(GVOCAB, 128)
    deq = pl.pallas_call(
        _deq_body,
        grid=(GVOCAB // GBLK,),
        in_specs=[pl.BlockSpec((GBLK, 128), lambda i: (i, 0))],
        out_specs=pl.BlockSpec((GBLK, 128), lambda i: (i, 0)),
        out_shape=jax.ShapeDtypeStruct((GVOCAB, 128), jnp.float32),
    )(tg)
    return deq.reshape(VOCAB, DIM)


@functools.partial(
    pl.kernel,
    out_type=jax.ShapeDtypeStruct((NBAGS, DIM), jnp.float32),
    mesh=plsc.VectorSubcoreMesh(core_axis_name="c", subcore_axis_name="s"),
    compiler_params=pltpu.CompilerParams(use_tc_tiling_on_sc=False, needs_layout_passes=False),
    scratch_types=[
        pltpu.VMEM((BAGS_W + 16,), jnp.int32),
        pltpu.VMEM((CH,), jnp.int32),
        pltpu.VMEM((CH,), jnp.int32),
        pltpu.VMEM((CH, DIM), jnp.float32),
        pltpu.VMEM((CH, DIM), jnp.float32),
        pltpu.VMEM((BAGS_W, DIM), jnp.float32),
        pltpu.VMEM((DIM,), jnp.float32),
        pltpu.SMEM((8,), jnp.int32),
        pltpu.SemaphoreType.DMA,
        pltpu.SemaphoreType.DMA,
    ],
)
def _bag_kernel(offs_hbm, idx_hbm, deq_hbm, out_hbm,
                offs_v, idx_a, idx_b, rows_a, rows_b, out_v, acc_v,
                bcur_s, sem_a, sem_b):
    wid = lax.axis_index("s") * 2 + lax.axis_index("c")
    bag_lo = wid * BAGS_W
    pltpu.sync_copy(offs_hbm.at[pl.ds(bag_lo, BAGS_W + 16)], offs_v)
    start = offs_v[pl.ds(0, 16)][0]
    end = offs_v[pl.ds(BAGS_W, 16)][0]
    abase = (start // 8) * 8
    nch = (end - abase + CH - 1) // CH

    zero = jnp.zeros((DIM,), jnp.float32)
    acc_v[...] = zero
    bcur_s[0] = jnp.int32(0)

    def _zero_body(i, _):
        out_v[i, :] = zero
        return 0
    lax.fori_loop(0, BAGS_W, _zero_body, 0)

    def _fetch(ci, idx_v, rows_v, sem):
        cb = abase + ci * CH
        pltpu.sync_copy(idx_hbm.at[pl.ds(cb, CH)], idx_v)
        for j in range(NGSUB):
            pltpu.make_async_copy(
                deq_hbm.at[idx_v.at[pl.ds(j * 128, 128)]],
                rows_v.at[pl.ds(j * 128, 128)], sem).start()

    def _drain(idx_v, rows_v, sem):
        for j in range(NGSUB):
            pltpu.make_async_copy(
                deq_hbm.at[idx_v.at[pl.ds(j * 128, 128)]],
                rows_v.at[pl.ds(j * 128, 128)], sem).wait()

    def _walk(ci, rows_v):
        cb = abase + ci * CH
        ce = cb + CH

        def _bstep(i, cnt):
            nc = cnt + (BAGS_W >> i)
            pi = jnp.minimum(nc, BAGS_W)
            probe = offs_v[pl.ds(pi, 16)][0]
            ok = jnp.logical_and(nc <= BAGS_W, probe <= ce)
            return jnp.where(ok, nc, cnt)
        cnt = lax.fori_loop(0, 10, _bstep, jnp.int32(0))

        bcur0 = bcur_s[0]
        nb_closed = jnp.maximum(cnt - bcur0, 0)
        extra = jnp.where(bcur0 + nb_closed < BAGS_W, 1, 0)
        trips = nb_closed + extra

        def _deq_row(x):
            mn = jnp.min(x)
            mx = jnp.max(x)
            scale = (mx - mn) * jnp.float32(1.0 / 255.0)
            scale = jnp.where(scale <= 0.0, jnp.float32(1e-8), scale)
            y = (x - mn) / scale + 0.5
            q = jnp.minimum(y.astype(jnp.int32).astype(jnp.float32), 255.0)
            return q * scale + mn

        def _bag_body(t, b):
            bv = offs_v[pl.ds(b, 16)]
            bs = bv[0]
            be = bv[1]
            ps = jnp.maximum(bs, cb)
            pe = jnp.minimum(be, ce)
            n = pe - ps
            l0 = ps - cb

            def _add8(j, accs):
                p = l0 + j * 8
                return tuple(accs[i] + _deq_row(rows_v[p + i, :])
                             for i in range(8))
            accs = lax.fori_loop(0, n >> 3, _add8,
                                 (zero,) * 8)
            t8 = ((accs[0] + accs[1]) + (accs[2] + accs[3])) + \
                 ((accs[4] + accs[5]) + (accs[6] + accs[7]))

            def _add1(p, a):
                return a + _deq_row(rows_v[p, :])
            tail = lax.fori_loop(l0 + (n & ~7), l0 + n, _add1, t8)

            total = acc_v[...] + tail
            closed = be <= ce

            @pl.when(closed)
            def _():
                out_v[b, :] = total
                acc_v[...] = zero

            @pl.when(jnp.logical_not(closed))
            def _():
                acc_v[...] = total

            return jnp.where(closed, b + 1, b)

        bcur_s[0] = lax.fori_loop(0, trips, _bag_body, bcur0)

    @pl.when(nch > 0)
    def _():
        _fetch(0, idx_a, rows_a, sem_a)

    def _pair_body(t, _):
        ci0 = 2 * t
        ci1 = ci0 + 1

        @pl.when(ci0 < nch)
        def _():
            @pl.when(ci1 < nch)
            def _():
                _fetch(ci1, idx_b, rows_b, sem_b)
            _drain(idx_a, rows_a, sem_a)
            _walk(ci0, rows_a)

        @pl.when(ci1 < nch)
        def _():
            @pl.when(ci1 + 1 < nch)
            def _():
                _fetch(ci1 + 1, idx_a, rows_a, sem_a)
            _drain(idx_b, rows_b, sem_b)
            _walk(ci1, rows_b)

        return 0

    lax.fori_loop(0, (nch + 1) // 2, _pair_body, 0)
    pltpu.sync_copy(out_v, out_hbm.at[pl.ds(bag_lo, BAGS_W)])


def kernel(table, indices, offsets):
    offs_p = jnp.concatenate(
        [offsets.astype(jnp.int32),
         jnp.full((OFFS_PAD - (NBAGS + 1),), NTOT, jnp.int32)])
    idx_p = jnp.concatenate(
        [indices.astype(jnp.int32), jnp.zeros((IDX_PAD - NTOT,), jnp.int32)])
    return _bag_kernel(offs_p, idx_p, table)
